# baseline (speedup 1.0000x reference)
import functools

import jax
import jax.numpy as jnp
from jax import lax
from jax.experimental import pallas as pl
from jax.experimental.pallas import tpu as pltpu, tpu_sc as plsc

N = 10000
E = 160000
DN = 256
DE = 16
LAT = 512
HW = 128

NC = 2
NS = 16
NW = NC * NS
CHUNK = 128
TOTC_E = E // CHUNK
KA = 62
KB = 18
TOTC = NS * (KA + KB)
E_PAD = TOTC * CHUNK
STRIPE = 8 * (-(-N // (NS * 8)))
N_ACC = NS * STRIPE

BE = 4000
BN = 512
N_PAD = -(-N // BN) * BN


def _sc_gather_scatter(node_v, idx2, didx2, ef, zeros_z):
    mesh = plsc.VectorSubcoreMesh(core_axis_name="c", subcore_axis_name="s")

    @functools.partial(
        pl.kernel,
        mesh=mesh,
        compiler_params=pltpu.CompilerParams(use_tc_tiling_on_sc=False),
        out_type=(
            jax.ShapeDtypeStruct((E_PAD, HW), jnp.float32),
            jax.ShapeDtypeStruct((E_PAD, HW), jnp.float32),
            jax.ShapeDtypeStruct((NC, N_ACC, DE), jnp.float32),
        ),
        scratch_types=[
            pltpu.VMEM((KA, CHUNK), jnp.int32),
            pltpu.VMEM((KA, CHUNK), jnp.int32),
            pltpu.VMEM((4, CHUNK, HW), jnp.float32),
            pltpu.VMEM((CHUNK, DE), jnp.float32),
            pltpu.VMEM((CHUNK, DE), jnp.float32),
            pltpu.VMEM((STRIPE, DE), jnp.float32),
            pltpu.VMEM_SHARED((N_ACC, DE), jnp.float32),
            pltpu.SemaphoreType.DMA,
            pltpu.SemaphoreType.DMA,
            pltpu.SemaphoreType.DMA,
            pltpu.SemaphoreType.DMA,
            pltpu.SemaphoreType.DMA,
        ],
    )
    def kern(node_hbm, idx_hbm, didx_hbm, edge_hbm, zeros_hbm,
             gsrc_hbm, gdst_hbm, psum_hbm,
             idx_v, didx_v, rows, erows, erows2, zbuf, acc,
             sg0, sg1, sw0, sw1, sem_z):
        c = lax.axis_index("c")
        s = lax.axis_index("s")
        cbase = jnp.where(c == 0, s * KA, NS * KA + s * KB)
        kw = jnp.where(c == 0, KA, KB)

        pltpu.sync_copy(idx_hbm.at[pl.ds(cbase, KA)], idx_v)
        pltpu.sync_copy(didx_hbm.at[pl.ds(cbase, KA)], didx_v)
        pltpu.async_copy(zeros_hbm, zbuf, sem_z).wait()
        pltpu.sync_copy(zbuf, acc.at[pl.ds(s * STRIPE, STRIPE)])
        plsc.subcore_barrier()

        gsems = (sg0, sg1)
        wsems = (sw0, sw1)
        outs = (gsrc_hbm, gdst_hbm)

        srcdst = (idx_v, didx_v)
        ebufs = (erows, erows2)

        @pl.loop(0, kw, step=2)
        def _loop(j0):
            gs = []
            for u in range(2):
                jc = j0 + u
                for p in range(2):
                    gs.append(pltpu.async_copy(
                        node_hbm.at[srcdst[p].at[jc]],
                        rows.at[2 * u + p], gsems[p]))
            ecs = []
            for u in range(2):
                g = cbase + j0 + u

                @pl.when(g < TOTC_E)
                def _eload(g=g, u=u):
                    ecs.append(pltpu.async_copy(
                        edge_hbm.at[pl.ds(g * CHUNK, CHUNK)], ebufs[u],
                        sem_z))
            ws = []
            for u in range(2):
                jc = j0 + u
                off = (cbase + jc) * CHUNK
                for p in range(2):
                    gs[2 * u + p].wait()
                    ws.append(pltpu.async_copy(
                        rows.at[2 * u + p], outs[p].at[pl.ds(off, CHUNK)],
                        wsems[p]))
            for u in range(2):
                g = cbase + j0 + u

                @pl.when(g < TOTC_E)
                def _scat(g=g, u=u):
                    pltpu.make_async_copy(
                        edge_hbm.at[pl.ds(g * CHUNK, CHUNK)], ebufs[u],
                        sem_z).wait()
                    pltpu.sync_copy(ebufs[u], acc.at[didx_v.at[j0 + u]],
                                    add=True)
            for w in ws:
                w.wait()

        plsc.subcore_barrier()
        pltpu.sync_copy(acc.at[pl.ds(s * STRIPE, STRIPE)], zbuf)
        pltpu.sync_copy(zbuf, psum_hbm.at[c, pl.ds(s * STRIPE, STRIPE)])

    return kern(node_v, idx2, didx2, ef, zeros_z)


def _edge_mlp(gsrc, gdst, ef, ws, wd, w1x, w2, g, b):
    def body(gs_r, gd_r, ef_r, ws_r, wd_r, w1x_r, w2_r, g_r, b_r, out):
        ef32 = ef_r[...]
        bf = jnp.bfloat16
        f32 = jnp.float32
        gsrc_b = pltpu.bitcast(gs_r[...], bf).reshape(BE, DN)
        gdst_b = pltpu.bitcast(gd_r[...], bf).reshape(BE, DN)
        h = jnp.dot(gsrc_b, ws_r[...], preferred_element_type=f32)
        h = h + jnp.dot(gdst_b, wd_r[...], preferred_element_type=f32)
        h = h + jnp.dot(ef32.astype(bf), w1x_r[...], preferred_element_type=f32)
        h = h * jax.nn.sigmoid(h)
        u = jnp.dot(h.astype(bf), w2_r[...], preferred_element_type=f32)
        mu = jnp.mean(u, axis=-1, keepdims=True)
        var = jnp.mean((u - mu) * (u - mu), axis=-1, keepdims=True)
        y = (u - mu) * lax.rsqrt(var + 1e-5) * g_r[...] + b_r[...]
        out[...] = y + ef32

    grid = (E // BE,)
    return pl.pallas_call(
        body,
        grid=grid,
        in_specs=[
            pl.BlockSpec((BE, HW), lambda i: (i, 0)),
            pl.BlockSpec((BE, HW), lambda i: (i, 0)),
            pl.BlockSpec((BE, DE), lambda i: (i, 0)),
            pl.BlockSpec((DN, LAT), lambda i: (0, 0)),
            pl.BlockSpec((DN, LAT), lambda i: (0, 0)),
            pl.BlockSpec((DE, LAT), lambda i: (0, 0)),
            pl.BlockSpec((LAT, DE), lambda i: (0, 0)),
            pl.BlockSpec((1, DE), lambda i: (0, 0)),
            pl.BlockSpec((1, DE), lambda i: (0, 0)),
        ],
        out_specs=pl.BlockSpec((BE, DE), lambda i: (i, 0)),
        out_shape=jax.ShapeDtypeStruct((E, DE), jnp.float32),
    )(gsrc, gdst, ef, ws, wd, w1x, w2, g, b)


def _node_mlp(nf_pad, p0, p1, w1nn, w1ne, w2, g, b):
    def body(nf, p0_r, p1_r, w1nn_r, w1ne_r, w2_r, g_r, b_r, out):
        nf32 = nf[...]
        bf = jnp.bfloat16
        f32 = jnp.float32
        se = p0_r[...] + p1_r[...]
        h = jnp.dot(nf32.astype(bf), w1nn_r[...], preferred_element_type=f32)
        h = h + jnp.dot(se.astype(bf), w1ne_r[...], preferred_element_type=f32)
        h = h * jax.nn.sigmoid(h)
        u = jnp.dot(h.astype(bf), w2_r[...], preferred_element_type=f32)
        mu = jnp.mean(u, axis=-1, keepdims=True)
        var = jnp.mean((u - mu) * (u - mu), axis=-1, keepdims=True)
        y = (u - mu) * lax.rsqrt(var + 1e-5) * g_r[...] + b_r[...]
        out[...] = y + nf32

    grid = (N_PAD // BN,)
    return pl.pallas_call(
        body,
        grid=grid,
        in_specs=[
            pl.BlockSpec((BN, DN), lambda i: (i, 0)),
            pl.BlockSpec((BN, DE), lambda i: (i, 0)),
            pl.BlockSpec((BN, DE), lambda i: (i, 0)),
            pl.BlockSpec((DN, LAT), lambda i: (0, 0)),
            pl.BlockSpec((DE, LAT), lambda i: (0, 0)),
            pl.BlockSpec((LAT, DN), lambda i: (0, 0)),
            pl.BlockSpec((1, DN), lambda i: (0, 0)),
            pl.BlockSpec((1, DN), lambda i: (0, 0)),
        ],
        out_specs=pl.BlockSpec((BN, DN), lambda i: (i, 0)),
        out_shape=jax.ShapeDtypeStruct((N_PAD, DN), jnp.float32),
    )(nf_pad, p0, p1, w1nn, w1ne, w2, g, b)


def kernel(node_feats, edge_feats, src_idx, dst_idx,
           W1e, W2e, ge, be, W1n, W2n, gn, bn):
    nf = node_feats[0]
    ef = edge_feats[0]
    nf_bf = nf.astype(jnp.bfloat16)
    node_v = lax.bitcast_convert_type(nf_bf.reshape(N, HW, 2), jnp.float32)

    sidx = jnp.concatenate([src_idx, jnp.zeros((E_PAD - E,), jnp.int32)])
    didx = jnp.concatenate([dst_idx, jnp.zeros((E_PAD - E,), jnp.int32)])
    zpad = jnp.zeros((KA, CHUNK), jnp.int32)
    sidx2 = jnp.concatenate([sidx.reshape(TOTC, CHUNK), zpad], axis=0)
    didx2 = jnp.concatenate([didx.reshape(TOTC, CHUNK), zpad], axis=0)
    zeros_z = jnp.zeros((STRIPE, DE), jnp.float32)

    gsrc, gdst, psum = _sc_gather_scatter(node_v, sidx2, didx2, ef, zeros_z)

    bf = jnp.bfloat16
    perm = jnp.concatenate([jnp.arange(0, DN, 2, dtype=jnp.int32),
                            jnp.arange(1, DN, 2, dtype=jnp.int32)])
    out_e = _edge_mlp(
        gsrc, gdst, ef,
        W1e[:DN][perm].astype(bf), W1e[DN:2 * DN][perm].astype(bf),
        W1e[2 * DN:].astype(bf),
        W2e.astype(bf), ge.reshape(1, DE), be.reshape(1, DE))

    nf_pad = jnp.concatenate(
        [nf, jnp.zeros((N_PAD - N, DN), jnp.float32)], axis=0)
    p0 = jnp.concatenate(
        [psum[0, :N], jnp.zeros((N_PAD - N, DE), jnp.float32)], axis=0)
    p1 = jnp.concatenate(
        [psum[1, :N], jnp.zeros((N_PAD - N, DE), jnp.float32)], axis=0)

    out_n = _node_mlp(
        nf_pad, p0, p1,
        W1n[:DN].astype(bf), W1n[DN:].astype(bf),
        W2n.astype(bf), gn.reshape(1, DN), bn.reshape(1, DN))

    return (out_n[:N][None], out_e[None])


# speedup vs baseline: 1.2772x; 1.2772x over previous
import functools

import jax
import jax.numpy as jnp
from jax import lax
from jax.experimental import pallas as pl
from jax.experimental.pallas import tpu as pltpu, tpu_sc as plsc

N = 10000
E = 160000
DN = 256
DE = 16
LAT = 512
HW = 128

NC = 2
NS = 16
NW = NC * NS
CHUNK = 128
TOTC_E = E // CHUNK
KA = 60
KB = 19
TOTC = NS * (KA + KB)
E_PAD = TOTC * CHUNK
STRIPE = 8 * (-(-N // (NS * 8)))
N_ACC = NS * STRIPE

BE = 4000
BN = 512
N_PAD = -(-N // BN) * BN


def _sc_gather_scatter(node_v, idx2, didx2, ef, zeros_z):
    mesh = plsc.VectorSubcoreMesh(core_axis_name="c", subcore_axis_name="s")

    @functools.partial(
        pl.kernel,
        mesh=mesh,
        compiler_params=pltpu.CompilerParams(use_tc_tiling_on_sc=False),
        out_type=(
            jax.ShapeDtypeStruct((E_PAD, HW), jnp.float32),
            jax.ShapeDtypeStruct((E_PAD, HW), jnp.float32),
            jax.ShapeDtypeStruct((NC, N_ACC, DE), jnp.float32),
        ),
        scratch_types=[
            pltpu.VMEM((KA, CHUNK), jnp.int32),
            pltpu.VMEM((KA, CHUNK), jnp.int32),
            pltpu.VMEM((4, CHUNK, HW), jnp.float32),
            pltpu.VMEM((CHUNK, DE), jnp.float32),
            pltpu.VMEM((CHUNK, DE), jnp.float32),
            pltpu.VMEM((STRIPE, DE), jnp.float32),
            pltpu.VMEM_SHARED((N_ACC, DE), jnp.float32),
            pltpu.SemaphoreType.DMA,
            pltpu.SemaphoreType.DMA,
            pltpu.SemaphoreType.DMA,
            pltpu.SemaphoreType.DMA,
            pltpu.SemaphoreType.DMA,
        ],
    )
    def kern(node_hbm, idx_hbm, didx_hbm, edge_hbm, zeros_hbm,
             gsrc_hbm, gdst_hbm, psum_hbm,
             idx_v, didx_v, rows, erows, erows2, zbuf, acc,
             sg0, sg1, sw0, sw1, sem_z):
        c = lax.axis_index("c")
        s = lax.axis_index("s")
        cbase = jnp.where(c == 0, s * KA, NS * KA + s * KB)
        kw = jnp.where(c == 0, KA, KB)

        pltpu.sync_copy(idx_hbm.at[pl.ds(cbase, KA)], idx_v)
        pltpu.sync_copy(didx_hbm.at[pl.ds(cbase, KA)], didx_v)
        pltpu.async_copy(zeros_hbm, zbuf, sem_z).wait()
        pltpu.sync_copy(zbuf, acc.at[pl.ds(s * STRIPE, STRIPE)])
        plsc.subcore_barrier()

        gsems = (sg0, sg1)
        wsems = (sw0, sw1)
        outs = (gsrc_hbm, gdst_hbm)

        srcdst = (idx_v, didx_v)

        @pl.loop(0, kw)
        def _loop(jc):
            g = cbase + jc
            off = g * CHUNK
            real = g < TOTC_E
            gs = [pltpu.async_copy(node_hbm.at[srcdst[p].at[jc]],
                                   rows.at[p], gsems[p])
                  for p in range(2)]

            @pl.when(real)
            def _eload():
                pltpu.async_copy(edge_hbm.at[pl.ds(off, CHUNK)], erows,
                                 sem_z).wait()

            ws = []
            for p in range(2):
                gs[p].wait()
                ws.append(pltpu.async_copy(
                    rows.at[p], outs[p].at[pl.ds(off, CHUNK)], wsems[p]))

            @pl.when(real)
            def _scat():
                pltpu.sync_copy(erows, acc.at[didx_v.at[jc]], add=True)

            for w in ws:
                w.wait()

        plsc.subcore_barrier()
        pltpu.sync_copy(acc.at[pl.ds(s * STRIPE, STRIPE)], zbuf)
        pltpu.sync_copy(zbuf, psum_hbm.at[c, pl.ds(s * STRIPE, STRIPE)])

    return kern(node_v, idx2, didx2, ef, zeros_z)


def _edge_mlp(gsrc, gdst, ef, ws, wd, w1x, w2, g, b):
    def body(gs_r, gd_r, ef_r, ws_r, wd_r, w1x_r, w2_r, g_r, b_r, out):
        ef32 = ef_r[...]
        bf = jnp.bfloat16
        f32 = jnp.float32
        gsrc_b = pltpu.bitcast(gs_r[...], bf).reshape(BE, DN)
        gdst_b = pltpu.bitcast(gd_r[...], bf).reshape(BE, DN)
        h = jnp.dot(gsrc_b, ws_r[...], preferred_element_type=f32)
        h = h + jnp.dot(gdst_b, wd_r[...], preferred_element_type=f32)
        h = h + jnp.dot(ef32.astype(bf), w1x_r[...], preferred_element_type=f32)
        h = h * jax.nn.sigmoid(h)
        u = jnp.dot(h.astype(bf), w2_r[...], preferred_element_type=f32)
        mu = jnp.mean(u, axis=-1, keepdims=True)
        var = jnp.mean((u - mu) * (u - mu), axis=-1, keepdims=True)
        y = (u - mu) * lax.rsqrt(var + 1e-5) * g_r[...] + b_r[...]
        out[...] = y + ef32

    grid = (E // BE,)
    return pl.pallas_call(
        body,
        grid=grid,
        in_specs=[
            pl.BlockSpec((BE, HW), lambda i: (i, 0)),
            pl.BlockSpec((BE, HW), lambda i: (i, 0)),
            pl.BlockSpec((BE, DE), lambda i: (i, 0)),
            pl.BlockSpec((DN, LAT), lambda i: (0, 0)),
            pl.BlockSpec((DN, LAT), lambda i: (0, 0)),
            pl.BlockSpec((DE, LAT), lambda i: (0, 0)),
            pl.BlockSpec((LAT, DE), lambda i: (0, 0)),
            pl.BlockSpec((1, DE), lambda i: (0, 0)),
            pl.BlockSpec((1, DE), lambda i: (0, 0)),
        ],
        out_specs=pl.BlockSpec((BE, DE), lambda i: (i, 0)),
        out_shape=jax.ShapeDtypeStruct((E, DE), jnp.float32),
    )(gsrc, gdst, ef, ws, wd, w1x, w2, g, b)


def _node_mlp(nf_pad, p0, p1, w1nn, w1ne, w2, g, b):
    def body(nf, p0_r, p1_r, w1nn_r, w1ne_r, w2_r, g_r, b_r, out):
        nf32 = nf[...]
        bf = jnp.bfloat16
        f32 = jnp.float32
        se = p0_r[...] + p1_r[...]
        h = jnp.dot(nf32.astype(bf), w1nn_r[...], preferred_element_type=f32)
        h = h + jnp.dot(se.astype(bf), w1ne_r[...], preferred_element_type=f32)
        h = h * jax.nn.sigmoid(h)
        u = jnp.dot(h.astype(bf), w2_r[...], preferred_element_type=f32)
        mu = jnp.mean(u, axis=-1, keepdims=True)
        var = jnp.mean((u - mu) * (u - mu), axis=-1, keepdims=True)
        y = (u - mu) * lax.rsqrt(var + 1e-5) * g_r[...] + b_r[...]
        out[...] = y + nf32

    grid = (N_PAD // BN,)
    return pl.pallas_call(
        body,
        grid=grid,
        in_specs=[
            pl.BlockSpec((BN, DN), lambda i: (i, 0)),
            pl.BlockSpec((BN, DE), lambda i: (i, 0)),
            pl.BlockSpec((BN, DE), lambda i: (i, 0)),
            pl.BlockSpec((DN, LAT), lambda i: (0, 0)),
            pl.BlockSpec((DE, LAT), lambda i: (0, 0)),
            pl.BlockSpec((LAT, DN), lambda i: (0, 0)),
            pl.BlockSpec((1, DN), lambda i: (0, 0)),
            pl.BlockSpec((1, DN), lambda i: (0, 0)),
        ],
        out_specs=pl.BlockSpec((BN, DN), lambda i: (i, 0)),
        out_shape=jax.ShapeDtypeStruct((N_PAD, DN), jnp.float32),
    )(nf_pad, p0, p1, w1nn, w1ne, w2, g, b)


def kernel(node_feats, edge_feats, src_idx, dst_idx,
           W1e, W2e, ge, be, W1n, W2n, gn, bn):
    nf = node_feats[0]
    ef = edge_feats[0]
    nf_bf = nf.astype(jnp.bfloat16)
    node_v = lax.bitcast_convert_type(nf_bf.reshape(N, HW, 2), jnp.float32)

    sidx = jnp.concatenate([src_idx, jnp.zeros((E_PAD - E,), jnp.int32)])
    didx = jnp.concatenate([dst_idx, jnp.zeros((E_PAD - E,), jnp.int32)])
    zpad = jnp.zeros((KA, CHUNK), jnp.int32)
    sidx2 = jnp.concatenate([sidx.reshape(TOTC, CHUNK), zpad], axis=0)
    didx2 = jnp.concatenate([didx.reshape(TOTC, CHUNK), zpad], axis=0)
    zeros_z = jnp.zeros((STRIPE, DE), jnp.float32)

    gsrc, gdst, psum = _sc_gather_scatter(node_v, sidx2, didx2, ef, zeros_z)

    bf = jnp.bfloat16
    perm = jnp.concatenate([jnp.arange(0, DN, 2, dtype=jnp.int32),
                            jnp.arange(1, DN, 2, dtype=jnp.int32)])
    out_e = _edge_mlp(
        gsrc, gdst, ef,
        W1e[:DN][perm].astype(bf), W1e[DN:2 * DN][perm].astype(bf),
        W1e[2 * DN:].astype(bf),
        W2e.astype(bf), ge.reshape(1, DE), be.reshape(1, DE))

    nf_pad = jnp.concatenate(
        [nf, jnp.zeros((N_PAD - N, DN), jnp.float32)], axis=0)
    p0 = jnp.concatenate(
        [psum[0, :N], jnp.zeros((N_PAD - N, DE), jnp.float32)], axis=0)
    p1 = jnp.concatenate(
        [psum[1, :N], jnp.zeros((N_PAD - N, DE), jnp.float32)], axis=0)

    out_n = _node_mlp(
        nf_pad, p0, p1,
        W1n[:DN].astype(bf), W1n[DN:].astype(bf),
        W2n.astype(bf), gn.reshape(1, DN), bn.reshape(1, DN))

    return (out_n[:N][None], out_e[None])


# speedup vs baseline: 1.4842x; 1.1620x over previous
import functools

import jax
import jax.numpy as jnp
from jax import lax
from jax.experimental import pallas as pl
from jax.experimental.pallas import tpu as pltpu, tpu_sc as plsc

N = 10000
E = 160000
DN = 256
DE = 16
LAT = 512
HW = 128

NC = 2
NS = 16
NW = NC * NS
CHUNK = 128
TOTC_E = E // CHUNK
KA = 60
KB = 19
TOTC = NS * (KA + KB)
E_PAD = TOTC * CHUNK
STRIPE = 8 * (-(-N // (NS * 8)))
N_ACC = NS * STRIPE

BE = 4000
BN = 512
N_PAD = -(-N // BN) * BN


def _sc_gather_scatter(node_v, idx2, didx2, ef, zeros_z):
    mesh = plsc.VectorSubcoreMesh(core_axis_name="c", subcore_axis_name="s")

    @functools.partial(
        pl.kernel,
        mesh=mesh,
        compiler_params=pltpu.CompilerParams(use_tc_tiling_on_sc=False),
        out_type=(
            jax.ShapeDtypeStruct((E_PAD, HW), jnp.float32),
            jax.ShapeDtypeStruct((E_PAD, HW), jnp.float32),
            jax.ShapeDtypeStruct((NC, N_ACC, DE), jnp.float32),
        ),
        scratch_types=[
            pltpu.VMEM((KA, CHUNK), jnp.int32),
            pltpu.VMEM((KA, CHUNK), jnp.int32),
            pltpu.VMEM((2, CHUNK, HW), jnp.float32),
            pltpu.VMEM((CHUNK, DE), jnp.float32),
            pltpu.VMEM((STRIPE, DE), jnp.float32),
            pltpu.VMEM_SHARED((N_ACC, DE), jnp.float32),
            pltpu.SemaphoreType.DMA,
            pltpu.SemaphoreType.DMA,
            pltpu.SemaphoreType.DMA,
            pltpu.SemaphoreType.DMA,
            pltpu.SemaphoreType.DMA,
        ],
    )
    def kern(node_hbm, idx_hbm, didx_hbm, edge_hbm, zeros_hbm,
             gsrc_hbm, gdst_hbm, psum_hbm,
             idx_v, didx_v, rows, erows, zbuf, acc,
             sg0, sg1, sw0, sw1, sem_z):
        c = lax.axis_index("c")
        s = lax.axis_index("s")
        cbase = jnp.where(c == 0, s * KA, NS * KA + s * KB)
        kw = jnp.where(c == 0, KA, KB)

        pltpu.sync_copy(idx_hbm.at[pl.ds(cbase, KA)], idx_v)
        pltpu.sync_copy(didx_hbm.at[pl.ds(cbase, KA)], didx_v)
        pltpu.async_copy(zeros_hbm, zbuf, sem_z).wait()
        pltpu.sync_copy(zbuf, acc.at[pl.ds(s * STRIPE, STRIPE)])
        plsc.subcore_barrier()

        gsems = (sg0, sg1)
        wsems = (sw0, sw1)
        outs = (gsrc_hbm, gdst_hbm)

        srcdst = (idx_v, didx_v)

        @pl.loop(0, kw)
        def _loop(jc):
            g = cbase + jc
            off = g * CHUNK
            real = g < TOTC_E
            gs = [pltpu.async_copy(node_hbm.at[srcdst[p].at[jc]],
                                   rows.at[p], gsems[p])
                  for p in range(2)]

            @pl.when(real)
            def _eload():
                pltpu.async_copy(edge_hbm.at[pl.ds(off, CHUNK)], erows,
                                 sem_z).wait()

            ws = []
            for p in range(2):
                gs[p].wait()
                ws.append(pltpu.async_copy(
                    rows.at[p], outs[p].at[pl.ds(off, CHUNK)], wsems[p]))

            @pl.when(real)
            def _scat():
                pltpu.sync_copy(erows, acc.at[didx_v.at[jc]], add=True)

            for w in ws:
                w.wait()

        plsc.subcore_barrier()
        pltpu.sync_copy(acc.at[pl.ds(s * STRIPE, STRIPE)], zbuf)
        pltpu.sync_copy(zbuf, psum_hbm.at[c, pl.ds(s * STRIPE, STRIPE)])

    return kern(node_v, idx2, didx2, ef, zeros_z)


def _edge_mlp(gsrc, gdst, ef, ws, wd, w1x, w2, g, b):
    def body(gs_r, gd_r, ef_r, ws_r, wd_r, w1x_r, w2_r, g_r, b_r, out):
        ef32 = ef_r[...]
        bf = jnp.bfloat16
        f32 = jnp.float32
        gsrc_b = pltpu.bitcast(gs_r[...], bf).reshape(BE, DN)
        gdst_b = pltpu.bitcast(gd_r[...], bf).reshape(BE, DN)
        h = jnp.dot(gsrc_b, ws_r[...], preferred_element_type=f32)
        h = h + jnp.dot(gdst_b, wd_r[...], preferred_element_type=f32)
        h = h + jnp.dot(ef32.astype(bf), w1x_r[...], preferred_element_type=f32)
        h = h * jax.nn.sigmoid(h)
        u = jnp.dot(h.astype(bf), w2_r[...], preferred_element_type=f32)
        mu = jnp.mean(u, axis=-1, keepdims=True)
        var = jnp.mean((u - mu) * (u - mu), axis=-1, keepdims=True)
        y = (u - mu) * lax.rsqrt(var + 1e-5) * g_r[...] + b_r[...]
        out[...] = y + ef32

    grid = (E // BE,)
    return pl.pallas_call(
        body,
        grid=grid,
        in_specs=[
            pl.BlockSpec((BE, HW), lambda i: (i, 0)),
            pl.BlockSpec((BE, HW), lambda i: (i, 0)),
            pl.BlockSpec((BE, DE), lambda i: (i, 0)),
            pl.BlockSpec((DN, LAT), lambda i: (0, 0)),
            pl.BlockSpec((DN, LAT), lambda i: (0, 0)),
            pl.BlockSpec((DE, LAT), lambda i: (0, 0)),
            pl.BlockSpec((LAT, DE), lambda i: (0, 0)),
            pl.BlockSpec((1, DE), lambda i: (0, 0)),
            pl.BlockSpec((1, DE), lambda i: (0, 0)),
        ],
        out_specs=pl.BlockSpec((BE, DE), lambda i: (i, 0)),
        out_shape=jax.ShapeDtypeStruct((E, DE), jnp.float32),
    )(gsrc, gdst, ef, ws, wd, w1x, w2, g, b)


def _node_mlp(nf_pad, p0, p1, w1nn, w1ne, w2, g, b):
    def body(nf, p0_r, p1_r, w1nn_r, w1ne_r, w2_r, g_r, b_r, out):
        nf32 = nf[...]
        bf = jnp.bfloat16
        f32 = jnp.float32
        se = p0_r[...] + p1_r[...]
        h = jnp.dot(nf32.astype(bf), w1nn_r[...], preferred_element_type=f32)
        h = h + jnp.dot(se.astype(bf), w1ne_r[...], preferred_element_type=f32)
        h = h * jax.nn.sigmoid(h)
        u = jnp.dot(h.astype(bf), w2_r[...], preferred_element_type=f32)
        mu = jnp.mean(u, axis=-1, keepdims=True)
        var = jnp.mean((u - mu) * (u - mu), axis=-1, keepdims=True)
        y = (u - mu) * lax.rsqrt(var + 1e-5) * g_r[...] + b_r[...]
        out[...] = y + nf32

    grid = (N_PAD // BN,)
    return pl.pallas_call(
        body,
        grid=grid,
        in_specs=[
            pl.BlockSpec((BN, DN), lambda i: (i, 0)),
            pl.BlockSpec((BN, DE), lambda i: (i, 0)),
            pl.BlockSpec((BN, DE), lambda i: (i, 0)),
            pl.BlockSpec((DN, LAT), lambda i: (0, 0)),
            pl.BlockSpec((DE, LAT), lambda i: (0, 0)),
            pl.BlockSpec((LAT, DN), lambda i: (0, 0)),
            pl.BlockSpec((1, DN), lambda i: (0, 0)),
            pl.BlockSpec((1, DN), lambda i: (0, 0)),
        ],
        out_specs=pl.BlockSpec((BN, DN), lambda i: (i, 0)),
        out_shape=jax.ShapeDtypeStruct((N_PAD, DN), jnp.float32),
    )(nf_pad, p0, p1, w1nn, w1ne, w2, g, b)


def kernel(node_feats, edge_feats, src_idx, dst_idx,
           W1e, W2e, ge, be, W1n, W2n, gn, bn):
    nf = node_feats[0]
    ef = edge_feats[0]
    nf_bf = nf.astype(jnp.bfloat16)
    node_v = lax.bitcast_convert_type(
        jnp.stack([nf_bf[:, :HW], nf_bf[:, HW:]], axis=-1), jnp.float32)

    sidx = jnp.concatenate([src_idx, jnp.zeros((E_PAD - E,), jnp.int32)])
    didx = jnp.concatenate([dst_idx, jnp.zeros((E_PAD - E,), jnp.int32)])
    zpad = jnp.zeros((KA, CHUNK), jnp.int32)
    sidx2 = jnp.concatenate([sidx.reshape(TOTC, CHUNK), zpad], axis=0)
    didx2 = jnp.concatenate([didx.reshape(TOTC, CHUNK), zpad], axis=0)
    zeros_z = jnp.zeros((STRIPE, DE), jnp.float32)

    gsrc, gdst, psum = _sc_gather_scatter(node_v, sidx2, didx2, ef, zeros_z)

    bf = jnp.bfloat16
    out_e = _edge_mlp(
        gsrc, gdst, ef,
        W1e[:DN].astype(bf), W1e[DN:2 * DN].astype(bf),
        W1e[2 * DN:].astype(bf),
        W2e.astype(bf), ge.reshape(1, DE), be.reshape(1, DE))

    nf_pad = jnp.concatenate(
        [nf, jnp.zeros((N_PAD - N, DN), jnp.float32)], axis=0)
    p0 = jnp.concatenate(
        [psum[0, :N], jnp.zeros((N_PAD - N, DE), jnp.float32)], axis=0)
    p1 = jnp.concatenate(
        [psum[1, :N], jnp.zeros((N_PAD - N, DE), jnp.float32)], axis=0)

    out_n = _node_mlp(
        nf_pad, p0, p1,
        W1n[:DN].astype(bf), W1n[DN:].astype(bf),
        W2n.astype(bf), gn.reshape(1, DN), bn.reshape(1, DN))

    return (out_n[:N][None], out_e[None])
